# baseline (speedup 1.0000x reference)
import functools

import jax
import jax.numpy as jnp
from jax import lax
from jax.experimental import pallas as pl
from jax.experimental.pallas import tpu as pltpu
from jax.experimental.pallas import tpu_sc as plsc

EMBEDDING_DIM = 32
CHUNK = 512
NBUF = 2
KSUB = 2
SUB = CHUNK // KSUB
BBN = CHUNK // 128
CELEM = CHUNK * EMBEDDING_DIM


@jax.jit
def _embedding_lookup(idx_flat, table):
    info = plsc.get_sparse_core_info()
    num_workers = info.num_cores * info.num_subcores
    b_total = idx_flat.shape[0]
    b_per_w = b_total // num_workers
    n_chunks = b_per_w // CHUNK
    bb_total = b_total // 128
    cg_stride = bb_total * 1024

    mesh = plsc.VectorSubcoreMesh(core_axis_name="c", subcore_axis_name="s")

    @functools.partial(
        pl.kernel,
        mesh=mesh,
        out_type=jax.ShapeDtypeStruct((4 * cg_stride,), jnp.float32),
        scratch_types=[
            pltpu.VMEM((b_per_w,), jnp.int32),
            pltpu.VMEM((NBUF, CHUNK, EMBEDDING_DIM), jnp.float32),
            pltpu.VMEM((NBUF * CELEM,), jnp.float32),
        ]
        + [pltpu.SemaphoreType.DMA] * (2 * NBUF),
        compiler_params=pltpu.CompilerParams(
            use_tc_tiling_on_sc=False, needs_layout_passes=False
        ),
    )
    def emb_kernel(idx_hbm, table_hbm, out_x, idx_v, rows_v, xp_v, *sems):
        gsem = sems[:NBUF]
        ssem = sems[NBUF:]
        wid = lax.axis_index("s") * info.num_cores + lax.axis_index("c")
        base = wid * b_per_w
        bb_base = wid * (b_per_w // 128)
        lane = lax.iota(jnp.int32, 16)
        col_vecs = [jnp.full((16,), cs, jnp.int32) for cs in range(32)]

        def gather_start(g, b):
            for k in range(KSUB):
                pltpu.async_copy(
                    table_hbm.at[idx_v.at[pl.ds(g * CHUNK + k * SUB, SUB)]],
                    rows_v.at[b].at[pl.ds(k * SUB, SUB)],
                    gsem[b],
                )

        def gather_wait(g, b):
            for k in range(KSUB):
                pltpu.make_async_copy(
                    table_hbm.at[idx_v.at[pl.ds(g * CHUNK + k * SUB, SUB)]],
                    rows_v.at[b].at[pl.ds(k * SUB, SUB)],
                    gsem[b],
                ).wait()

        def transpose_chunk(b):
            rows = rows_v.at[b]

            @pl.loop(0, BBN * 8)
            def _xp(k):
                bbj = k >> 3
                l0 = (k & 7) * 16
                dyn = b * CELEM + bbj * 1024 + l0
                rvec = bbj * 128 + l0 + lane
                for cs in range(32):
                    vals = plsc.load_gather(rows, [rvec, col_vecs[cs]])
                    off = ((cs >> 3) * BBN * 1024) + ((cs & 7) * 128)
                    xp_v[pl.ds(dyn + off, 16)] = vals

        def store_issue(g, b):
            bb0 = bb_base + g * BBN
            for cg in range(4):
                pltpu.async_copy(
                    xp_v.at[pl.ds(b * CELEM + cg * (BBN * 1024), BBN * 1024)],
                    out_x.at[pl.ds(cg * cg_stride + bb0 * 1024, BBN * 1024)],
                    ssem[b],
                )

        def store_wait(g, b):
            bb0 = bb_base + g * BBN
            for cg in range(4):
                pltpu.make_async_copy(
                    xp_v.at[pl.ds(b * CELEM + cg * (BBN * 1024), BBN * 1024)],
                    out_x.at[pl.ds(cg * cg_stride + bb0 * 1024, BBN * 1024)],
                    ssem[b],
                ).wait()

        pltpu.sync_copy(idx_hbm.at[pl.ds(base, b_per_w)], idx_v)

        gather_start(0, 0)
        gather_start(1, 1)

        def chunk_step(g, b, drain_prev, issue_ahead):
            gather_wait(g, b)
            if drain_prev:
                store_wait(g - NBUF, b)
            transpose_chunk(b)
            if issue_ahead:
                gather_start(g + NBUF, b)
            store_issue(g, b)

        chunk_step(0, 0, False, True)
        chunk_step(1, 1, False, True)

        @pl.loop(2, n_chunks - NBUF, step=NBUF)
        def _steady(outer):
            for j in range(NBUF):
                chunk_step(outer + j, j, True, True)

        for gg in range(n_chunks - NBUF, n_chunks):
            chunk_step(gg, gg % NBUF, True, False)
        for gg in range(n_chunks - NBUF, n_chunks):
            store_wait(gg, gg % NBUF)

    return emb_kernel(idx_flat, table)


def kernel(x, table):
    idx_flat = x.reshape(-1).astype(jnp.int32)
    xo = _embedding_lookup(idx_flat, table)
    x4 = xo.reshape(4, x.size // 128, 8, 128)
    out2d = x4.transpose(0, 2, 1, 3).reshape(EMBEDDING_DIM, x.size).T
    return out2d.reshape(x.shape + (EMBEDDING_DIM,))


# speedup vs baseline: 1.1709x; 1.1709x over previous
import functools

import jax
import jax.numpy as jnp
from jax import lax
from jax.experimental import pallas as pl
from jax.experimental.pallas import tpu as pltpu
from jax.experimental.pallas import tpu_sc as plsc

EMBEDDING_DIM = 32
CHUNK = 512
NBUF = 2
KSUB = 2
SUB = CHUNK // KSUB
BBN = CHUNK // 128
CELEM = CHUNK * EMBEDDING_DIM


@jax.jit
def _embedding_lookup(idx_flat, table):
    info = plsc.get_sparse_core_info()
    num_workers = info.num_cores * info.num_subcores
    b_total = idx_flat.shape[0]
    b_per_w = b_total // num_workers
    n_chunks = b_per_w // CHUNK
    bb_total = b_total // 128
    cg_stride = bb_total * 1024

    mesh = plsc.VectorSubcoreMesh(core_axis_name="c", subcore_axis_name="s")

    @functools.partial(
        pl.kernel,
        mesh=mesh,
        out_type=jax.ShapeDtypeStruct((4 * cg_stride,), jnp.float32),
        scratch_types=[
            pltpu.VMEM((b_per_w,), jnp.int32),
            pltpu.VMEM((NBUF, CHUNK, EMBEDDING_DIM), jnp.float32),
            pltpu.VMEM((NBUF * CELEM,), jnp.float32),
        ]
        + [pltpu.SemaphoreType.DMA] * (2 * NBUF),
        compiler_params=pltpu.CompilerParams(
            use_tc_tiling_on_sc=False, needs_layout_passes=False
        ),
    )
    def emb_kernel(idx_hbm, table_hbm, out_x, idx_v, rows_v, xp_v, *sems):
        gsem = sems[:NBUF]
        ssem = sems[NBUF:]
        wid = lax.axis_index("s") * info.num_cores + lax.axis_index("c")
        base = wid * b_per_w
        bb_base = wid * (b_per_w // 128)
        lane = lax.iota(jnp.int32, 16)
        col_vecs = [jnp.full((16,), cs, jnp.int32) for cs in range(32)]

        def gather_start(g, b):
            for k in range(KSUB):
                pltpu.async_copy(
                    table_hbm.at[idx_v.at[pl.ds(g * CHUNK + k * SUB, SUB)]],
                    rows_v.at[b].at[pl.ds(k * SUB, SUB)],
                    gsem[b],
                )

        def gather_wait(g, b):
            for k in range(KSUB):
                pltpu.make_async_copy(
                    table_hbm.at[idx_v.at[pl.ds(g * CHUNK + k * SUB, SUB)]],
                    rows_v.at[b].at[pl.ds(k * SUB, SUB)],
                    gsem[b],
                ).wait()

        def transpose_chunk(b):
            rows = rows_v.at[b]

            @pl.loop(0, BBN * 8)
            def _xp(k):
                bbj = k >> 3
                l0 = (k & 7) * 16
                dyn = b * CELEM + bbj * 1024 + l0
                rvec = bbj * 128 + l0 + lane
                vals = [
                    plsc.load_gather(rows, [rvec, col_vecs[cs]])
                    for cs in range(32)
                ]
                for cs in range(32):
                    off = ((cs >> 3) * BBN * 1024) + ((cs & 7) * 128)
                    xp_v[pl.ds(dyn + off, 16)] = vals[cs]

        def store_issue(g, b):
            bb0 = bb_base + g * BBN
            for cg in range(4):
                pltpu.async_copy(
                    xp_v.at[pl.ds(b * CELEM + cg * (BBN * 1024), BBN * 1024)],
                    out_x.at[pl.ds(cg * cg_stride + bb0 * 1024, BBN * 1024)],
                    ssem[b],
                )

        def store_wait(g, b):
            bb0 = bb_base + g * BBN
            for cg in range(4):
                pltpu.make_async_copy(
                    xp_v.at[pl.ds(b * CELEM + cg * (BBN * 1024), BBN * 1024)],
                    out_x.at[pl.ds(cg * cg_stride + bb0 * 1024, BBN * 1024)],
                    ssem[b],
                ).wait()

        pltpu.sync_copy(idx_hbm.at[pl.ds(base, b_per_w)], idx_v)

        gather_start(0, 0)
        gather_start(1, 1)

        def chunk_step(g, b, drain_prev, issue_ahead):
            gather_wait(g, b)
            if drain_prev:
                store_wait(g - NBUF, b)
            transpose_chunk(b)
            if issue_ahead:
                gather_start(g + NBUF, b)
            store_issue(g, b)

        chunk_step(0, 0, False, True)
        chunk_step(1, 1, False, True)

        @pl.loop(2, n_chunks - NBUF, step=NBUF)
        def _steady(outer):
            for j in range(NBUF):
                chunk_step(outer + j, j, True, True)

        for gg in range(n_chunks - NBUF, n_chunks):
            chunk_step(gg, gg % NBUF, True, False)
        for gg in range(n_chunks - NBUF, n_chunks):
            store_wait(gg, gg % NBUF)

    return emb_kernel(idx_flat, table)


def kernel(x, table):
    idx_flat = x.reshape(-1).astype(jnp.int32)
    xo = _embedding_lookup(idx_flat, table)
    x4 = xo.reshape(4, x.size // 128, 8, 128)
    out2d = x4.transpose(0, 2, 1, 3).reshape(EMBEDDING_DIM, x.size).T
    return out2d.reshape(x.shape + (EMBEDDING_DIM,))


# speedup vs baseline: 1.4292x; 1.2206x over previous
import functools

import jax
import jax.numpy as jnp
from jax import lax
from jax.experimental import pallas as pl
from jax.experimental.pallas import tpu as pltpu
from jax.experimental.pallas import tpu_sc as plsc

EMBEDDING_DIM = 32
CHUNK = 512
NBUF = 2
KSUB = 2
SUB = CHUNK // KSUB
BBN = CHUNK // 128
CELEM = CHUNK * EMBEDDING_DIM


@jax.jit
def _embedding_lookup(idx_flat, table):
    info = plsc.get_sparse_core_info()
    num_workers = info.num_cores * info.num_subcores
    b_total = idx_flat.shape[0]
    b_per_w = b_total // num_workers
    n_chunks = b_per_w // CHUNK
    bb_total = b_total // 128
    cg_stride = bb_total * 1024

    mesh = plsc.VectorSubcoreMesh(core_axis_name="c", subcore_axis_name="s")

    @functools.partial(
        pl.kernel,
        mesh=mesh,
        out_type=jax.ShapeDtypeStruct((4 * cg_stride,), jnp.float32),
        scratch_types=[
            pltpu.VMEM((b_per_w,), jnp.int32),
            pltpu.VMEM((NBUF, CHUNK, EMBEDDING_DIM), jnp.float32),
            pltpu.VMEM((NBUF * CELEM,), jnp.float32),
        ]
        + [pltpu.SemaphoreType.DMA] * (2 * NBUF),
        compiler_params=pltpu.CompilerParams(
            use_tc_tiling_on_sc=False, needs_layout_passes=False
        ),
    )
    def emb_kernel(idx_hbm, table_hbm, out_x, idx_v, rows_v, xp_v, *sems):
        gsem = sems[:NBUF]
        ssem = sems[NBUF:]
        wid = lax.axis_index("s") * info.num_cores + lax.axis_index("c")
        base = wid * b_per_w
        bb_base = wid * (b_per_w // 128)
        lane = lax.iota(jnp.int32, 16)
        col_vecs = [jnp.full((16,), cs, jnp.int32) for cs in range(32)]

        def gather_start(g, b):
            for k in range(KSUB):
                pltpu.async_copy(
                    table_hbm.at[idx_v.at[pl.ds(g * CHUNK + k * SUB, SUB)]],
                    rows_v.at[b].at[pl.ds(k * SUB, SUB)],
                    gsem[b],
                )

        def gather_wait(g, b):
            for k in range(KSUB):
                pltpu.make_async_copy(
                    table_hbm.at[idx_v.at[pl.ds(g * CHUNK + k * SUB, SUB)]],
                    rows_v.at[b].at[pl.ds(k * SUB, SUB)],
                    gsem[b],
                ).wait()

        def transpose_chunk(b):
            rows = rows_v.at[b]

            @functools.partial(plsc.parallel_loop, 0, BBN * 8)
            def _xp(k):
                bbj = k >> 3
                l0 = (k & 7) * 16
                dyn = b * CELEM + bbj * 1024 + l0
                rvec = bbj * 128 + l0 + lane
                vals = [
                    plsc.load_gather(rows, [rvec, col_vecs[cs]])
                    for cs in range(32)
                ]
                for cs in range(32):
                    off = ((cs >> 3) * BBN * 1024) + ((cs & 7) * 128)
                    xp_v[pl.ds(dyn + off, 16)] = vals[cs]

            plsc.subcore_barrier()

        def store_issue(g, b):
            bb0 = bb_base + g * BBN
            for cg in range(4):
                pltpu.async_copy(
                    xp_v.at[pl.ds(b * CELEM + cg * (BBN * 1024), BBN * 1024)],
                    out_x.at[pl.ds(cg * cg_stride + bb0 * 1024, BBN * 1024)],
                    ssem[b],
                )

        def store_wait(g, b):
            bb0 = bb_base + g * BBN
            for cg in range(4):
                pltpu.make_async_copy(
                    xp_v.at[pl.ds(b * CELEM + cg * (BBN * 1024), BBN * 1024)],
                    out_x.at[pl.ds(cg * cg_stride + bb0 * 1024, BBN * 1024)],
                    ssem[b],
                ).wait()

        pltpu.sync_copy(idx_hbm.at[pl.ds(base, b_per_w)], idx_v)

        gather_start(0, 0)
        gather_start(1, 1)

        def chunk_step(g, b, drain_prev, issue_ahead):
            gather_wait(g, b)
            if drain_prev:
                store_wait(g - NBUF, b)
            transpose_chunk(b)
            if issue_ahead:
                gather_start(g + NBUF, b)
            store_issue(g, b)

        chunk_step(0, 0, False, True)
        chunk_step(1, 1, False, True)

        @pl.loop(2, n_chunks - NBUF, step=NBUF)
        def _steady(outer):
            for j in range(NBUF):
                chunk_step(outer + j, j, True, True)

        for gg in range(n_chunks - NBUF, n_chunks):
            chunk_step(gg, gg % NBUF, True, False)
        for gg in range(n_chunks - NBUF, n_chunks):
            store_wait(gg, gg % NBUF)

    return emb_kernel(idx_flat, table)


def kernel(x, table):
    idx_flat = x.reshape(-1).astype(jnp.int32)
    xo = _embedding_lookup(idx_flat, table)
    x4 = xo.reshape(4, x.size // 128, 8, 128)
    out2d = x4.transpose(0, 2, 1, 3).reshape(EMBEDDING_DIM, x.size).T
    return out2d.reshape(x.shape + (EMBEDDING_DIM,))


# speedup vs baseline: 1.4329x; 1.0026x over previous
import functools

import jax
import jax.numpy as jnp
from jax import lax
from jax.experimental import pallas as pl
from jax.experimental.pallas import tpu as pltpu
from jax.experimental.pallas import tpu_sc as plsc

EMBEDDING_DIM = 32
CHUNK = 512
NBUF = 2
KSUB = 2
SUB = CHUNK // KSUB
BBN = CHUNK // 128
CELEM = CHUNK * EMBEDDING_DIM


@jax.jit
def _embedding_lookup(idx_flat, table):
    info = plsc.get_sparse_core_info()
    num_workers = info.num_cores * info.num_subcores
    b_total = idx_flat.shape[0]
    b_per_w = b_total // num_workers
    n_chunks = b_per_w // CHUNK
    bb_total = b_total // 128
    cg_stride = bb_total * 1024

    mesh = plsc.VectorSubcoreMesh(core_axis_name="c", subcore_axis_name="s")

    @functools.partial(
        pl.kernel,
        mesh=mesh,
        out_type=jax.ShapeDtypeStruct((4 * cg_stride,), jnp.float32),
        scratch_types=[
            pltpu.VMEM((b_per_w,), jnp.int32),
            pltpu.VMEM((NBUF, CHUNK, EMBEDDING_DIM), jnp.float32),
            pltpu.VMEM((NBUF * CELEM,), jnp.float32),
        ]
        + [pltpu.SemaphoreType.DMA] * (2 * NBUF),
        compiler_params=pltpu.CompilerParams(
            use_tc_tiling_on_sc=False, needs_layout_passes=False
        ),
    )
    def emb_kernel(idx_hbm, table_hbm, out_x, idx_v, rows_v, xp_v, *sems):
        gsem = sems[:NBUF]
        ssem = sems[NBUF:]
        wid = lax.axis_index("s") * info.num_cores + lax.axis_index("c")
        base = wid * b_per_w
        bb_base = wid * (b_per_w // 128)
        lane = lax.iota(jnp.int32, 16)
        col_vecs = [jnp.full((16,), cs, jnp.int32) for cs in range(32)]

        def gather_start(g, b):
            for k in range(KSUB):
                pltpu.async_copy(
                    table_hbm.at[idx_v.at[pl.ds(g * CHUNK + k * SUB, SUB)]],
                    rows_v.at[b].at[pl.ds(k * SUB, SUB)],
                    gsem[b],
                )

        def gather_wait(g, b):
            for k in range(KSUB):
                pltpu.make_async_copy(
                    table_hbm.at[idx_v.at[pl.ds(g * CHUNK + k * SUB, SUB)]],
                    rows_v.at[b].at[pl.ds(k * SUB, SUB)],
                    gsem[b],
                ).wait()

        def transpose_chunk(b):
            rows = rows_v.at[b]

            @functools.partial(plsc.parallel_loop, 0, BBN * 8, unroll=2)
            def _xp(k):
                bbj = k >> 3
                l0 = (k & 7) * 16
                dyn = b * CELEM + bbj * 1024 + l0
                rvec = bbj * 128 + l0 + lane
                vals = [
                    plsc.load_gather(rows, [rvec, col_vecs[cs]])
                    for cs in range(32)
                ]
                for cs in range(32):
                    off = ((cs >> 3) * BBN * 1024) + ((cs & 7) * 128)
                    xp_v[pl.ds(dyn + off, 16)] = vals[cs]

            plsc.subcore_barrier()

        def store_issue(g, b):
            bb0 = bb_base + g * BBN
            for cg in range(4):
                pltpu.async_copy(
                    xp_v.at[pl.ds(b * CELEM + cg * (BBN * 1024), BBN * 1024)],
                    out_x.at[pl.ds(cg * cg_stride + bb0 * 1024, BBN * 1024)],
                    ssem[b],
                )

        def store_wait(g, b):
            bb0 = bb_base + g * BBN
            for cg in range(4):
                pltpu.make_async_copy(
                    xp_v.at[pl.ds(b * CELEM + cg * (BBN * 1024), BBN * 1024)],
                    out_x.at[pl.ds(cg * cg_stride + bb0 * 1024, BBN * 1024)],
                    ssem[b],
                ).wait()

        pltpu.sync_copy(idx_hbm.at[pl.ds(base, b_per_w)], idx_v)

        gather_start(0, 0)
        gather_start(1, 1)

        def chunk_step(g, b, drain_prev, issue_ahead):
            gather_wait(g, b)
            if drain_prev:
                store_wait(g - NBUF, b)
            transpose_chunk(b)
            if issue_ahead:
                gather_start(g + NBUF, b)
            store_issue(g, b)

        chunk_step(0, 0, False, True)
        chunk_step(1, 1, False, True)

        @pl.loop(2, n_chunks - NBUF, step=NBUF)
        def _steady(outer):
            for j in range(NBUF):
                chunk_step(outer + j, j, True, True)

        for gg in range(n_chunks - NBUF, n_chunks):
            chunk_step(gg, gg % NBUF, True, False)
        for gg in range(n_chunks - NBUF, n_chunks):
            store_wait(gg, gg % NBUF)

    return emb_kernel(idx_flat, table)


def kernel(x, table):
    idx_flat = x.reshape(-1).astype(jnp.int32)
    xo = _embedding_lookup(idx_flat, table)
    x4 = xo.reshape(4, x.size // 128, 8, 128)
    out2d = x4.transpose(0, 2, 1, 3).reshape(EMBEDDING_DIM, x.size).T
    return out2d.reshape(x.shape + (EMBEDDING_DIM,))


# speedup vs baseline: 2.8142x; 1.9639x over previous
import functools

import jax
import jax.numpy as jnp
from jax import lax
from jax.experimental import pallas as pl
from jax.experimental.pallas import tpu as pltpu
from jax.experimental.pallas import tpu_sc as plsc

EMBEDDING_DIM = 32
NBUF = 2
ROWS = 128
CTILE = ROWS * EMBEDDING_DIM


@jax.jit
def _embedding_lookup(idx_flat, table):
    info = plsc.get_sparse_core_info()
    num_workers = info.num_cores * info.num_subcores
    b_total = idx_flat.shape[0]
    n_i, n_j = b_total // 50, 50
    b_per_w = b_total // num_workers
    i_per_w = n_i // num_workers
    ib_per_w = i_per_w // 128
    n_chunks = n_j * ib_per_w
    j_stride = 4 * (n_i // 128) * 1024
    cg_stride = (n_i // 128) * 1024

    mesh = plsc.VectorSubcoreMesh(core_axis_name="c", subcore_axis_name="s")

    @functools.partial(
        pl.kernel,
        mesh=mesh,
        out_type=jax.ShapeDtypeStruct((b_total * EMBEDDING_DIM,), jnp.float32),
        scratch_types=[
            pltpu.VMEM((b_per_w,), jnp.int32),
            pltpu.VMEM((b_per_w,), jnp.int32),
            pltpu.VMEM((NBUF, ROWS, EMBEDDING_DIM), jnp.float32),
            pltpu.VMEM((NBUF * CTILE,), jnp.float32),
        ]
        + [pltpu.SemaphoreType.DMA] * (2 * NBUF),
        compiler_params=pltpu.CompilerParams(
            use_tc_tiling_on_sc=False, needs_layout_passes=False
        ),
    )
    def emb_kernel(idx_hbm, table_hbm, out_x, idx_v, idxp_v, rows_v, xp_v, *sems):
        gsem = sems[:NBUF]
        ssem = sems[NBUF:]
        wid = lax.axis_index("s") * info.num_cores + lax.axis_index("c")
        base = wid * b_per_w
        lane = lax.iota(jnp.int32, 16)
        col_vecs = [jnp.full((16,), cs, jnp.int32) for cs in range(32)]

        def dst_off(q, cg):
            return (
                (q >> 2) * j_stride
                + cg * cg_stride
                + wid * (ib_per_w * 1024)
                + (q & 3) * 1024
            )

        def gather_start(q, b):
            pltpu.async_copy(
                table_hbm.at[idxp_v.at[pl.ds(q * ROWS, ROWS)]],
                rows_v.at[b],
                gsem[b],
            )

        def gather_wait(q, b):
            pltpu.make_async_copy(
                table_hbm.at[idxp_v.at[pl.ds(q * ROWS, ROWS)]],
                rows_v.at[b],
                gsem[b],
            ).wait()

        def transpose_chunk(b):
            rows = rows_v.at[b]

            @functools.partial(plsc.parallel_loop, 0, ROWS // 16)
            def _xp(k):
                l0 = k * 16
                rvec = l0 + lane
                vals = [
                    plsc.load_gather(rows, [rvec, col_vecs[cs]])
                    for cs in range(32)
                ]
                for cs in range(32):
                    off = b * CTILE + (cs >> 3) * 1024 + (cs & 7) * 128 + l0
                    xp_v[pl.ds(off, 16)] = vals[cs]

            plsc.subcore_barrier()

        def store_issue(q, b):
            for cg in range(4):
                pltpu.async_copy(
                    xp_v.at[pl.ds(b * CTILE + cg * 1024, 1024)],
                    out_x.at[pl.ds(dst_off(q, cg), 1024)],
                    ssem[b],
                )

        def store_wait(q, b):
            for cg in range(4):
                pltpu.make_async_copy(
                    xp_v.at[pl.ds(b * CTILE + cg * 1024, 1024)],
                    out_x.at[pl.ds(dst_off(q, cg), 1024)],
                    ssem[b],
                ).wait()

        pltpu.sync_copy(idx_hbm.at[pl.ds(base, b_per_w)], idx_v)

        @pl.loop(0, b_per_w // 16)
        def _perm(k):
            j = k // (i_per_w // 16)
            i0 = (k % (i_per_w // 16)) * 16
            src = plsc.load_gather(idx_v, [(i0 + lane) * n_j + j])
            idxp_v[pl.ds(j * i_per_w + i0, 16)] = src

        plsc.subcore_barrier()

        for b in range(NBUF):
            gather_start(b, b)

        def chunk_step(q, b, drain_prev, issue_ahead):
            gather_wait(q, b)
            if drain_prev:
                store_wait(q - NBUF, b)
            transpose_chunk(b)
            if issue_ahead:
                gather_start(q + NBUF, b)
            store_issue(q, b)

        for qq in range(NBUF):
            chunk_step(qq, qq, False, True)

        @pl.loop(NBUF, n_chunks - NBUF, step=NBUF)
        def _steady(outer):
            for j in range(NBUF):
                chunk_step(outer + j, j, True, True)

        for qq in range(n_chunks - NBUF, n_chunks):
            chunk_step(qq, qq % NBUF, True, False)
        for qq in range(n_chunks - NBUF, n_chunks):
            store_wait(qq, qq % NBUF)

    return emb_kernel(idx_flat, table)


def kernel(x, table):
    idx_flat = x.reshape(-1).astype(jnp.int32)
    xo = _embedding_lookup(idx_flat, table)
    x7 = xo.reshape(50, 4, x.shape[0] // 128, 8, 128)
    return x7.transpose(2, 4, 0, 1, 3).reshape(x.shape + (EMBEDDING_DIM,))


# speedup vs baseline: 3.0178x; 1.0724x over previous
import functools

import jax
import jax.numpy as jnp
from jax import lax
from jax.experimental import pallas as pl
from jax.experimental.pallas import tpu as pltpu
from jax.experimental.pallas import tpu_sc as plsc

EMBEDDING_DIM = 32
NBUF = 4
ROWS = 128
CTILE = ROWS * EMBEDDING_DIM


@jax.jit
def _embedding_lookup(idx_flat, table):
    info = plsc.get_sparse_core_info()
    num_workers = info.num_cores * info.num_subcores
    b_total = idx_flat.shape[0]
    n_i, n_j = b_total // 50, 50
    b_per_w = b_total // num_workers
    i_per_w = n_i // num_workers
    ib_per_w = i_per_w // 128
    n_chunks = n_j * ib_per_w
    j_stride = 4 * (n_i // 128) * 1024
    cg_stride = (n_i // 128) * 1024

    mesh = plsc.VectorSubcoreMesh(core_axis_name="c", subcore_axis_name="s")

    @functools.partial(
        pl.kernel,
        mesh=mesh,
        out_type=jax.ShapeDtypeStruct((b_total * EMBEDDING_DIM,), jnp.float32),
        scratch_types=[
            pltpu.VMEM((b_per_w,), jnp.int32),
            pltpu.VMEM((b_per_w,), jnp.int32),
            pltpu.VMEM((NBUF, ROWS, EMBEDDING_DIM), jnp.float32),
            pltpu.VMEM((NBUF * CTILE,), jnp.float32),
        ]
        + [pltpu.SemaphoreType.DMA] * (2 * NBUF),
        compiler_params=pltpu.CompilerParams(
            use_tc_tiling_on_sc=False, needs_layout_passes=False
        ),
    )
    def emb_kernel(idx_hbm, table_hbm, out_x, idx_v, idxp_v, rows_v, xp_v, *sems):
        gsem = sems[:NBUF]
        ssem = sems[NBUF:]
        wid = lax.axis_index("s") * info.num_cores + lax.axis_index("c")
        base = wid * b_per_w
        lane = lax.iota(jnp.int32, 16)
        col_vecs = [jnp.full((16,), cs, jnp.int32) for cs in range(32)]

        def dst_off(q, cg):
            return (
                (q >> 2) * j_stride
                + cg * cg_stride
                + wid * (ib_per_w * 1024)
                + (q & 3) * 1024
            )

        def gather_start(q, b):
            pltpu.async_copy(
                table_hbm.at[idxp_v.at[pl.ds(q * ROWS, ROWS)]],
                rows_v.at[b],
                gsem[b],
            )

        def gather_wait(q, b):
            pltpu.make_async_copy(
                table_hbm.at[idxp_v.at[pl.ds(q * ROWS, ROWS)]],
                rows_v.at[b],
                gsem[b],
            ).wait()

        def transpose_chunk(b):
            rows = rows_v.at[b]

            @functools.partial(plsc.parallel_loop, 0, ROWS // 16)
            def _xp(k):
                l0 = k * 16
                rvec = l0 + lane
                vals = [
                    plsc.load_gather(rows, [rvec, col_vecs[cs]])
                    for cs in range(32)
                ]
                for cs in range(32):
                    off = b * CTILE + (cs >> 3) * 1024 + (cs & 7) * 128 + l0
                    xp_v[pl.ds(off, 16)] = vals[cs]

            plsc.subcore_barrier()

        def store_issue(q, b):
            for cg in range(4):
                pltpu.async_copy(
                    xp_v.at[pl.ds(b * CTILE + cg * 1024, 1024)],
                    out_x.at[pl.ds(dst_off(q, cg), 1024)],
                    ssem[b],
                )

        def store_wait(q, b):
            for cg in range(4):
                pltpu.make_async_copy(
                    xp_v.at[pl.ds(b * CTILE + cg * 1024, 1024)],
                    out_x.at[pl.ds(dst_off(q, cg), 1024)],
                    ssem[b],
                ).wait()

        pltpu.sync_copy(idx_hbm.at[pl.ds(base, b_per_w)], idx_v)

        @pl.loop(0, b_per_w // 16)
        def _perm(k):
            j = k // (i_per_w // 16)
            i0 = (k % (i_per_w // 16)) * 16
            src = plsc.load_gather(idx_v, [(i0 + lane) * n_j + j])
            idxp_v[pl.ds(j * i_per_w + i0, 16)] = src

        plsc.subcore_barrier()

        for b in range(NBUF):
            gather_start(b, b)

        def chunk_step(q, b, drain_prev, issue_ahead):
            gather_wait(q, b)
            if drain_prev:
                store_wait(q - NBUF, b)
            transpose_chunk(b)
            if issue_ahead:
                gather_start(q + NBUF, b)
            store_issue(q, b)

        for qq in range(NBUF):
            chunk_step(qq, qq, False, True)

        @pl.loop(NBUF, n_chunks - NBUF, step=NBUF)
        def _steady(outer):
            for j in range(NBUF):
                chunk_step(outer + j, j, True, True)

        for qq in range(n_chunks - NBUF, n_chunks):
            chunk_step(qq, qq % NBUF, True, False)
        for qq in range(n_chunks - NBUF, n_chunks):
            store_wait(qq, qq % NBUF)

    return emb_kernel(idx_flat, table)


def kernel(x, table):
    idx_flat = x.reshape(-1).astype(jnp.int32)
    xo = _embedding_lookup(idx_flat, table)
    x7 = xo.reshape(50, 4, x.shape[0] // 128, 8, 128)
    return x7.transpose(2, 4, 0, 1, 3).reshape(x.shape + (EMBEDDING_DIM,))
